# baseline (speedup 1.0000x reference)
import functools

import jax
import jax.numpy as jnp
from jax import lax
from jax.experimental import pallas as pl
from jax.experimental.pallas import tpu as pltpu
from jax.experimental.pallas import tpu_sc as plsc

_IDX_CHUNK = 128


def _gather_sc(table, idx):
  batch, dim = idx.shape[0], table.shape[1]
  info = plsc.get_sparse_core_info()
  num_workers = info.num_cores * info.num_subcores
  b_per_w = batch // num_workers
  n_chunks = b_per_w // _IDX_CHUNK
  mesh = plsc.VectorSubcoreMesh(core_axis_name="c", subcore_axis_name="s")

  @functools.partial(
      pl.kernel,
      mesh=mesh,
      out_type=jax.ShapeDtypeStruct((batch, dim), jnp.float32),
      scratch_types=[
          pltpu.VMEM((b_per_w,), jnp.int32),
          pltpu.VMEM((b_per_w, dim), jnp.float32),
          pltpu.VMEM_SHARED((1000, 128), jnp.float32),
          pltpu.SemaphoreType.DMA((n_chunks,)),
          pltpu.SemaphoreType.DMA,
      ],
  )
  def gather_kernel(
      table_hbm, idx_hbm, out_hbm, idx_v, rows_v, tab_s, gsem, wsem
  ):
    sid = lax.axis_index("s")
    wid = sid * info.num_cores + lax.axis_index("c")
    base = wid * b_per_w
    idx_cp = pltpu.async_copy(idx_hbm.at[pl.ds(base, b_per_w)], idx_v, wsem)

    n_ld = 5
    rows_ld = tab_s.shape[0] // n_ld

    @pl.when(sid < n_ld)
    def _load_table():
      pltpu.sync_copy(
          table_hbm.at[pl.ds(sid * rows_ld, rows_ld)],
          tab_s.at[pl.ds(sid * rows_ld, rows_ld)],
      )

    idx_cp.wait()
    plsc.subcore_barrier()

    def start_gather(j):
      return pltpu.async_copy(
          tab_s.at[idx_v.at[pl.ds(j * _IDX_CHUNK, _IDX_CHUNK)]],
          rows_v.at[pl.ds(j * _IDX_CHUNK, _IDX_CHUNK)],
          gsem.at[j],
      )

    depth = 4
    gathers = [start_gather(j) for j in range(min(depth, n_chunks))]
    writes = []
    for j in range(n_chunks):
      gathers[j].wait()
      writes.append(
          pltpu.async_copy(
              rows_v.at[pl.ds(j * _IDX_CHUNK, _IDX_CHUNK)],
              out_hbm.at[pl.ds(base + j * _IDX_CHUNK, _IDX_CHUNK)],
              wsem,
          )
      )
      if j + depth < n_chunks:
        gathers.append(start_gather(j + depth))
    for w in writes:
      w.wait()

  return gather_kernel(table, idx)


_DN_T = (((1,), (1,)), ((), ()))


def _mlp_body(emb_ref, w1_ref, b1_ref, w2_ref, b2_ref, out_ref):
  x = emb_ref[...]
  h = (
      lax.dot_general(x, w1_ref[...], _DN_T, preferred_element_type=jnp.float32)
      + b1_ref[...]
  )
  h = h * jax.nn.sigmoid(h)
  out_ref[...] = (
      lax.dot_general(h, w2_ref[...], _DN_T, preferred_element_type=jnp.float32)
      + b2_ref[...]
  )


def _mlp_tc(emb, w1t, b1, w2t, b2, blk=2048):
  batch, dim = emb.shape
  grid = (batch // blk,)
  assert batch % blk == 0
  return pl.pallas_call(
      _mlp_body,
      grid=grid,
      in_specs=[
          pl.BlockSpec((blk, dim), lambda i: (i, 0)),
          pl.BlockSpec((dim, dim), lambda i: (0, 0)),
          pl.BlockSpec((1, dim), lambda i: (0, 0)),
          pl.BlockSpec((dim, dim), lambda i: (0, 0)),
          pl.BlockSpec((1, dim), lambda i: (0, 0)),
      ],
      out_specs=pl.BlockSpec((blk, dim), lambda i: (i, 0)),
      out_shape=jax.ShapeDtypeStruct((batch, dim), jnp.float32),
  )(emb, w1t, b1, w2t, b2)


def kernel(class_labels, table, W1, b1, W2, b2):
  idx = class_labels.astype(jnp.int32)
  mlp_table = _mlp_tc(
      table, W1, b1.reshape(1, -1), W2, b2.reshape(1, -1),
      blk=table.shape[0],
  )
  return _gather_sc(mlp_table, idx)


# speedup vs baseline: 1.0020x; 1.0020x over previous
import functools

import jax
import jax.numpy as jnp
from jax import lax
from jax.experimental import pallas as pl
from jax.experimental.pallas import tpu as pltpu
from jax.experimental.pallas import tpu_sc as plsc

_IDX_CHUNK = 128


def _gather_sc(table, idx):
  batch, dim = idx.shape[0], table.shape[1]
  info = plsc.get_sparse_core_info()
  num_workers = info.num_cores * info.num_subcores
  b_per_w = batch // num_workers
  n_chunks = b_per_w // _IDX_CHUNK
  mesh = plsc.VectorSubcoreMesh(core_axis_name="c", subcore_axis_name="s")

  @functools.partial(
      pl.kernel,
      mesh=mesh,
      out_type=jax.ShapeDtypeStruct((batch, dim), jnp.float32),
      scratch_types=[
          pltpu.VMEM((b_per_w,), jnp.int32),
          pltpu.VMEM((b_per_w, dim), jnp.float32),
          pltpu.VMEM_SHARED(table.shape, jnp.float32),
          pltpu.SemaphoreType.DMA((n_chunks,)),
          pltpu.SemaphoreType.DMA,
      ],
  )
  def gather_kernel(
      table_hbm, idx_hbm, out_hbm, idx_v, rows_v, tab_s, gsem, wsem
  ):
    sid = lax.axis_index("s")
    wid = sid * info.num_cores + lax.axis_index("c")
    base = wid * b_per_w
    idx_cp = pltpu.async_copy(idx_hbm.at[pl.ds(base, b_per_w)], idx_v, wsem)

    n_ld = 5
    rows_ld = tab_s.shape[0] // n_ld

    @pl.when(sid < n_ld)
    def _load_table():
      pltpu.sync_copy(
          table_hbm.at[pl.ds(sid * rows_ld, rows_ld)],
          tab_s.at[pl.ds(sid * rows_ld, rows_ld)],
      )

    idx_cp.wait()
    plsc.subcore_barrier()

    def start_gather(j):
      return pltpu.async_copy(
          tab_s.at[idx_v.at[pl.ds(j * _IDX_CHUNK, _IDX_CHUNK)]],
          rows_v.at[pl.ds(j * _IDX_CHUNK, _IDX_CHUNK)],
          gsem.at[j],
      )

    depth = 4
    gathers = [start_gather(j) for j in range(min(depth, n_chunks))]
    writes = []
    for j in range(n_chunks):
      gathers[j].wait()
      writes.append(
          pltpu.async_copy(
              rows_v.at[pl.ds(j * _IDX_CHUNK, _IDX_CHUNK)],
              out_hbm.at[pl.ds(base + j * _IDX_CHUNK, _IDX_CHUNK)],
              wsem,
          )
      )
      if j + depth < n_chunks:
        gathers.append(start_gather(j + depth))
    for w in writes:
      w.wait()

  return gather_kernel(table, idx)


_DN_T = (((1,), (1,)), ((), ()))


def _mlp_body(emb_ref, w1_ref, b1_ref, w2_ref, b2_ref, out_ref):
  x = emb_ref[...]
  h = (
      lax.dot_general(x, w1_ref[...], _DN_T, preferred_element_type=jnp.float32)
      + b1_ref[...]
  )
  h = h * jax.nn.sigmoid(h)
  out_ref[...] = (
      lax.dot_general(h, w2_ref[...], _DN_T, preferred_element_type=jnp.float32)
      + b2_ref[...]
  )


def _mlp_tc(emb, w1t, b1, w2t, b2, blk=2048):
  batch, dim = emb.shape
  grid = (batch // blk,)
  assert batch % blk == 0
  return pl.pallas_call(
      _mlp_body,
      grid=grid,
      in_specs=[
          pl.BlockSpec((blk, dim), lambda i: (i, 0)),
          pl.BlockSpec((dim, dim), lambda i: (0, 0)),
          pl.BlockSpec((1, dim), lambda i: (0, 0)),
          pl.BlockSpec((dim, dim), lambda i: (0, 0)),
          pl.BlockSpec((1, dim), lambda i: (0, 0)),
      ],
      out_specs=pl.BlockSpec((blk, dim), lambda i: (i, 0)),
      out_shape=jax.ShapeDtypeStruct((batch, dim), jnp.float32),
  )(emb, w1t, b1, w2t, b2)


def kernel(class_labels, table, W1, b1, W2, b2):
  idx = class_labels.astype(jnp.int32)
  mlp_table = _mlp_tc(
      table, W1, b1.reshape(1, -1), W2, b2.reshape(1, -1),
      blk=table.shape[0],
  )
  return _gather_sc(mlp_table, idx)
